# Initial kernel scaffold; baseline (speedup 1.0000x reference)
import jax
import jax.numpy as jnp
from jax.experimental import pallas as pl


def kernel(labels, features, preds, centers):
    raise NotImplementedError("write your pallas kernel here")


# speedup vs baseline: 2.4273x; 2.4273x over previous
import functools

import jax
import jax.numpy as jnp
from jax import lax
from jax.experimental import pallas as pl
from jax.experimental.pallas import tpu as pltpu
from jax.experimental.pallas import tpu_sc as plsc

N = 262144
D = 64
C = 85
L = 16
NC = 2
NS = 16
NW = NC * NS
ROWS_PER_W = N // NW
CHUNK = 128
N_CHUNKS = ROWS_PER_W // CHUNK


def _sc_body(labels_hbm, features_hbm, centers_hbm, out_hbm,
             idx_v, fbuf, cbuf, acc_v, sem):
    wid = lax.axis_index("s") * NC + lax.axis_index("c")
    base = wid * ROWS_PER_W

    def chunk_body(g, acc):
        off = base + g * CHUNK
        pltpu.sync_copy(labels_hbm.at[pl.ds(off, CHUNK)], idx_v)
        gat = pltpu.async_copy(centers_hbm.at[idx_v], cbuf, sem)
        pltpu.sync_copy(features_hbm.at[pl.ds(off, CHUNK), :], fbuf)
        gat.wait()

        def row_body(i, a):
            for k in range(D // L):
                f = fbuf[i, pl.ds(k * L, L)]
                c = cbuf[i, pl.ds(k * L, L)]
                d = f - c
                a = a + d * d
            return a

        return lax.fori_loop(0, CHUNK, row_body, acc)

    acc = lax.fori_loop(0, N_CHUNKS, chunk_body,
                        jnp.zeros((L,), jnp.float32))
    acc_v[...] = acc
    pltpu.sync_copy(acc_v, out_hbm.at[wid])


@jax.jit
def _center_loss(labels, features, centers):
    mesh = plsc.VectorSubcoreMesh(core_axis_name="c", subcore_axis_name="s")
    partials = pl.kernel(
        _sc_body,
        out_type=jax.ShapeDtypeStruct((NW, L), jnp.float32),
        mesh=mesh,
        scratch_types=[
            pltpu.VMEM((CHUNK,), jnp.int32),
            pltpu.VMEM((CHUNK, D), jnp.float32),
            pltpu.VMEM((CHUNK, D), jnp.float32),
            pltpu.VMEM((L,), jnp.float32),
            pltpu.SemaphoreType.DMA,
        ],
        compiler_params=pltpu.CompilerParams(use_tc_tiling_on_sc=False),
    )(labels, features, centers)
    return 0.5 * jnp.sum(partials)


def kernel(labels, features, preds, centers):
    del preds
    return _center_loss(labels, features, centers)
